# baseline (speedup 1.0000x reference)
import functools

import jax
import jax.numpy as jnp
from jax import lax
from jax.experimental import pallas as pl
from jax.experimental.pallas import tpu as pltpu
from jax.experimental.pallas import tpu_sc as plsc

BATCH = 16384
DIM = 64
LANES = 16
CHUNK = 128

_info = plsc.get_sparse_core_info()
_NC, _NS = _info.num_cores, _info.num_subcores
_NW = _NC * _NS
_BPW = BATCH // _NW
_NIDX = _BPW // CHUNK
_NGROUP = _BPW // LANES


def _unpack32(x32):
    return plsc.unpack(x32, format=plsc.PackFormat.INTERLEAVED)


def _gmf_body(user2, item2, ut, it, wb16, out,
              uidx_v, iidx_v, urows, irows, wb_v, out_v, tr_v,
              sem_u, sem_i):
    wid = lax.axis_index("s") * _NC + lax.axis_index("c")

    pltpu.sync_copy(user2.at[pl.ds(wid * _NIDX, _NIDX)], uidx_v)
    pltpu.sync_copy(item2.at[pl.ds(wid * _NIDX, _NIDX)], iidx_v)
    pltpu.sync_copy(wb16, wb_v)

    copies = []
    for c in range(_NIDX):
        copies.append(pltpu.async_copy(
            ut.at[uidx_v.at[c]], urows.at[pl.ds(c * CHUNK, CHUNK)], sem_u))
        copies.append(pltpu.async_copy(
            it.at[iidx_v.at[c]], irows.at[pl.ds(c * CHUNK, CHUNK)], sem_i))
    for cp in copies:
        cp.wait()

    wfs = []
    for j in range(2):
        wa, wb_ = _unpack32(wb_v[pl.ds(j * 32, 32)])
        wfs += [wa, wb_]
    bva, _ = _unpack32(wb_v[pl.ds(DIM, 32)])
    scat_idx = lax.iota(jnp.int32, LANES) * LANES

    def group(g, carry):
        for l in range(LANES):
            b = g * LANES + l
            acc = jnp.zeros((LANES,), jnp.float32)
            for j in range(2):
                ua, ub = _unpack32(urows[b, pl.ds(j * 32, 32)])
                ia, ib = _unpack32(irows[b, pl.ds(j * 32, 32)])
                acc = acc + ua * ia * wfs[2 * j]
                acc = acc + ub * ib * wfs[2 * j + 1]
            plsc.store_scatter(tr_v, [scat_idx + l], acc)
        tot = tr_v[pl.ds(0, LANES)]
        for l in range(1, LANES):
            tot = tot + tr_v[pl.ds(l * LANES, LANES)]
        x = tot + bva
        out_v[pl.ds(g * LANES, LANES)] = 1.0 / (1.0 + jnp.exp(-x))
        return carry

    lax.fori_loop(0, _NGROUP, group, 0)

    pltpu.sync_copy(out_v, out.at[pl.ds(wid * _BPW, _BPW)])


@jax.jit
def _gmf_sc(user2, item2, ut_bf, it_bf, wb16):
    mesh = plsc.VectorSubcoreMesh(core_axis_name="c", subcore_axis_name="s")
    run = functools.partial(
        pl.kernel,
        mesh=mesh,
        out_type=jax.ShapeDtypeStruct((BATCH,), jnp.float32),
        scratch_types=[
            pltpu.VMEM((_NIDX, CHUNK), jnp.int32),
            pltpu.VMEM((_NIDX, CHUNK), jnp.int32),
            pltpu.VMEM((_BPW, DIM), jnp.bfloat16),
            pltpu.VMEM((_BPW, DIM), jnp.bfloat16),
            pltpu.VMEM((DIM + 32,), jnp.bfloat16),
            pltpu.VMEM((_BPW,), jnp.float32),
            pltpu.VMEM((LANES * LANES,), jnp.float32),
            pltpu.SemaphoreType.DMA,
            pltpu.SemaphoreType.DMA,
        ],
        compiler_params=pltpu.CompilerParams(
            needs_layout_passes=False, use_tc_tiling_on_sc=False),
    )(_gmf_body)
    return run(user2, item2, ut_bf, it_bf, wb16)


def kernel(user, item, user_table, item_table, dense_w, dense_b):
    ut_bf = user_table.astype(jnp.bfloat16)
    it_bf = item_table.astype(jnp.bfloat16)
    wb16 = jnp.concatenate(
        [dense_w.reshape(DIM), jnp.broadcast_to(dense_b, (32,))]
    ).astype(jnp.bfloat16)
    user2 = user.astype(jnp.int32).reshape(_NW * _NIDX, CHUNK)
    item2 = item.astype(jnp.int32).reshape(_NW * _NIDX, CHUNK)
    return _gmf_sc(user2, item2, ut_bf, it_bf, wb16)


# speedup vs baseline: 2.6771x; 2.6771x over previous
import functools

import jax
import jax.numpy as jnp
from jax import lax
from jax.experimental import pallas as pl
from jax.experimental.pallas import tpu as pltpu
from jax.experimental.pallas import tpu_sc as plsc

BATCH = 16384
DIM = 64
LANES = 16
PANEL = 128
NFULL = 7812
TAILS = NFULL * PANEL
NIDXV = BATCH // LANES

_info = plsc.get_sparse_core_info()
_NC, _NS = _info.num_cores, _info.num_subcores
_NW = _NC * _NS
_PPW = -(-NFULL // _NW)
_BPW = BATCH // _NW
_NGROUP = _BPW // LANES
_HCAP = BATCH + 2 * LANES


def _gather_body(user_h, item_h, utT, itT, utail, itail, u_out, i_out,
                 idx_v, hit_r, hit_p, pb0, pb1, tb, rowbl,
                 semp0, semp1, semr):
    wid = lax.axis_index("s") * _NC + lax.axis_index("c")
    lo = wid * _PPW
    hi = jnp.minimum(lo + _PPW, NFULL)
    is_last = wid == (_NW - 1)
    hi_eff = hi + jnp.where(is_last, 1, 0)
    iota16 = lax.iota(jnp.int32, LANES)

    for idx_hbm, tT, tailT, st_out in ((user_h, utT, utail, u_out),
                                       (item_h, itT, itail, i_out)):
        pltpu.sync_copy(idx_hbm, idx_v)
        pltpu.sync_copy(tailT, tb)

        def scan(v, off):
            rvec = idx_v[pl.ds(v * LANES, LANES)]
            pan = lax.shift_right_arithmetic(rvec, 7)
            m = jnp.logical_and(pan >= lo, pan < hi_eff)
            cnt_v = plsc.all_reduce_population_count(m)
            dest = off + plsc.cumsum(jnp.where(m, 1, 0)) - 1
            plsc.store_scatter(hit_r, [dest], rvec, mask=m)
            plsc.store_scatter(hit_p, [dest], v * LANES + iota16, mask=m)
            return off + cnt_v[0]

        hn = lax.fori_loop(0, NIDXV, scan, 0)
        plsc.store_scatter(hit_r, [hn + iota16],
                           jnp.full((LANES,), -1, jnp.int32),
                           mask=iota16 < LANES)
        nv = lax.shift_right_arithmetic(hn + LANES - 1, 4)

        def fetch(p, buf, sem):
            off = pl.multiple_of(p * PANEL, PANEL)
            return pltpu.async_copy(tT.at[:, pl.ds(off, PANEL)], buf, sem)

        def drain_panel(buf, sem):
            pltpu.make_async_copy(
                tT.at[:, pl.ds(0, PANEL)], buf, sem).wait()

        def drain_rows(n):
            def w(_, c):
                pltpu.make_async_copy(
                    st_out.at[pl.ds(0, DIM)], rowbl.at[0], semr).wait()
                return c
            lax.fori_loop(0, n, w, 0)

        def extract(buf, hr, hp, pmi, vslot, width):
            for l in range(LANES):
                @pl.when(pmi[l] != 0)
                def _():
                    colv = jnp.broadcast_to(
                        jnp.bitwise_and(hr[l], PANEL - 1), (LANES,)
                    ).astype(jnp.int32)
                    slot = vslot * LANES + l
                    for j in range(DIM // LANES):
                        seg = plsc.load_gather(
                            buf, [iota16 + j * LANES, colv])
                        rowbl[slot, pl.ds(j * LANES, LANES)] = seg
                    dst = pl.multiple_of(hp[l] * DIM, DIM)
                    pltpu.async_copy(rowbl.at[slot],
                                     st_out.at[pl.ds(dst, DIM)], semr)

        def hits_for(p, buf, width):
            def hscan(v, c8):
                hr = hit_r[pl.ds(v * LANES, LANES)]
                hp = hit_p[pl.ds(v * LANES, LANES)]
                pm = lax.shift_right_arithmetic(hr, 7) == p
                pmi = jnp.where(pm, 1, 0)
                c = plsc.all_reduce_population_count(pm)[0]
                vslot = jnp.mod(v, 8)

                @pl.when(jnp.logical_and(vslot == 0, v > 0))
                def _():
                    drain_rows(c8)

                @pl.when(c > 0)
                def _():
                    extract(buf, hr, hp, pmi, vslot, width)
                return jnp.where(jnp.logical_and(vslot == 0, v > 0),
                                 c, c8 + c)

            c8f = lax.fori_loop(0, nv, hscan, 0)
            drain_rows(c8f)

        fetch(lo, pb0, semp0)

        @pl.when(lo + 1 < hi)
        def _():
            fetch(lo + 1, pb1, semp1)

        nk2 = lax.shift_right_arithmetic(hi_eff - lo + 1, 1)

        def panel_iter(k2, carry):
            for b, (pb, semp) in enumerate(((pb0, semp0), (pb1, semp1))):
                p = lo + 2 * k2 + b

                @pl.when(p < hi)
                def _(p=p, pb=pb, semp=semp):
                    drain_panel(pb, semp)
                    hits_for(p, pb, PANEL)

                    @pl.when(p + 2 < hi)
                    def _():
                        fetch(p + 2, pb, semp)

                @pl.when(jnp.logical_and(p == NFULL, is_last))
                def _(p=p):
                    hits_for(p, tb, DIM)
            return carry

        lax.fori_loop(0, nk2, panel_iter, 0)


def _dot_body(u_st, i_st, w64, b16, out,
              urows, irows, w_v, b_v, out_v, tr_v):
    wid = lax.axis_index("s") * _NC + lax.axis_index("c")
    base = wid * _BPW * DIM

    pltpu.sync_copy(u_st.at[pl.ds(base, _BPW * DIM)], urows)
    pltpu.sync_copy(i_st.at[pl.ds(base, _BPW * DIM)], irows)
    pltpu.sync_copy(w64, w_v)
    pltpu.sync_copy(b16, b_v)

    wvs = [w_v[pl.ds(j * LANES, LANES)] for j in range(DIM // LANES)]
    bv = b_v[...]
    scat_idx = lax.iota(jnp.int32, LANES) * LANES

    def group(g, carry):
        for l in range(LANES):
            b = g * LANES + l
            acc = jnp.zeros((LANES,), jnp.float32)
            for j in range(DIM // LANES):
                uv = urows[pl.ds(b * DIM + j * LANES, LANES)]
                iv = irows[pl.ds(b * DIM + j * LANES, LANES)]
                acc = acc + uv * iv * wvs[j]
            plsc.store_scatter(tr_v, [scat_idx + l], acc)
        tot = tr_v[pl.ds(0, LANES)]
        for l in range(1, LANES):
            tot = tot + tr_v[pl.ds(l * LANES, LANES)]
        x = tot + bv
        out_v[pl.ds(g * LANES, LANES)] = 1.0 / (1.0 + jnp.exp(-x))
        return carry

    lax.fori_loop(0, _NGROUP, group, 0)

    pltpu.sync_copy(out_v, out.at[pl.ds(wid * _BPW, _BPW)])


@jax.jit
def _gmf_sc(user, item, utT, itT, utail, itail, w64, b16):
    mesh = plsc.VectorSubcoreMesh(core_axis_name="c", subcore_axis_name="s")
    gather = functools.partial(
        pl.kernel,
        mesh=mesh,
        out_type=(jax.ShapeDtypeStruct((BATCH * DIM,), jnp.float32),
                  jax.ShapeDtypeStruct((BATCH * DIM,), jnp.float32)),
        scratch_types=[
            pltpu.VMEM((BATCH,), jnp.int32),
            pltpu.VMEM((_HCAP,), jnp.int32),
            pltpu.VMEM((_HCAP,), jnp.int32),
            pltpu.VMEM((DIM, PANEL), jnp.float32),
            pltpu.VMEM((DIM, PANEL), jnp.float32),
            pltpu.VMEM((DIM, DIM), jnp.float32),
            pltpu.VMEM((8 * LANES, DIM), jnp.float32),
            pltpu.SemaphoreType.DMA,
            pltpu.SemaphoreType.DMA,
            pltpu.SemaphoreType.DMA,
        ],
        compiler_params=pltpu.CompilerParams(needs_layout_passes=False),
    )(_gather_body)
    u_st, i_st = gather(user, item, utT, itT, utail, itail)

    dot = functools.partial(
        pl.kernel,
        mesh=mesh,
        out_type=jax.ShapeDtypeStruct((BATCH,), jnp.float32),
        scratch_types=[
            pltpu.VMEM((_BPW * DIM,), jnp.float32),
            pltpu.VMEM((_BPW * DIM,), jnp.float32),
            pltpu.VMEM((DIM,), jnp.float32),
            pltpu.VMEM((LANES,), jnp.float32),
            pltpu.VMEM((_BPW,), jnp.float32),
            pltpu.VMEM((LANES * LANES,), jnp.float32),
        ],
        compiler_params=pltpu.CompilerParams(needs_layout_passes=False),
    )(_dot_body)
    return dot(u_st, i_st, w64, b16)


def kernel(user, item, user_table, item_table, dense_w, dense_b):
    utT = user_table.T
    itT = item_table.T
    utail = user_table[TAILS:TAILS + DIM].T
    itail = item_table[TAILS:TAILS + DIM].T
    w64 = dense_w.reshape(DIM)
    b16 = jnp.broadcast_to(dense_b, (LANES,))
    return _gmf_sc(user.astype(jnp.int32), item.astype(jnp.int32),
                   utT, itT, utail, itail, w64, b16)


# speedup vs baseline: 3.1445x; 1.1746x over previous
import functools

import jax
import jax.numpy as jnp
from jax import lax
from jax.experimental import pallas as pl
from jax.experimental.pallas import tpu as pltpu
from jax.experimental.pallas import tpu_sc as plsc

BATCH = 16384
DIM = 64
LANES = 16
PANEL = 128
NFULL = 7812
TAILS = NFULL * PANEL
NIDXV = BATCH // LANES

_info = plsc.get_sparse_core_info()
_NC, _NS = _info.num_cores, _info.num_subcores
_NW = _NC * _NS
_PPW = -(-NFULL // _NW)
_BPW = BATCH // _NW
_NGROUP = _BPW // LANES
_HCAP = BATCH + 2 * LANES


def _gather_body(user_h, item_h, utT, itT, utail, itail, u_out, i_out,
                 idx_v, hit_r, hit_p, hit_r2, hit_p2, pb0, pb1, pb2, tb,
                 rowbl, soff, semp0, semp1, semp2, semr):
    wid = lax.axis_index("s") * _NC + lax.axis_index("c")
    lo = wid * _PPW
    hi = jnp.minimum(lo + _PPW, NFULL)
    is_last = wid == (_NW - 1)
    hi_eff = hi + jnp.where(is_last, 1, 0)
    iota16 = lax.iota(jnp.int32, LANES)

    for idx_hbm, tT, tailT, st_out in ((user_h, utT, utail, u_out),
                                       (item_h, itT, itail, i_out)):
        pltpu.sync_copy(idx_hbm, idx_v)
        pltpu.sync_copy(tailT, tb)

        def scan(v, off):
            rvec = idx_v[pl.ds(v * LANES, LANES)]
            pan = lax.shift_right_arithmetic(rvec, 7)
            m = jnp.logical_and(pan >= lo, pan < hi_eff)
            cnt_v = plsc.all_reduce_population_count(m)
            dest = off + plsc.cumsum(jnp.where(m, 1, 0)) - 1
            plsc.store_scatter(hit_r, [dest], rvec, mask=m)
            plsc.store_scatter(hit_p, [dest], v * LANES + iota16, mask=m)
            return off + cnt_v[0]

        hn = lax.fori_loop(0, NIDXV, scan, 0)
        nv = lax.shift_right_arithmetic(hn + LANES - 1, 4)

        off2 = 0
        for sub in range(16):
            soff[sub] = off2

            def rebin(v, o, sub=sub):
                hr = hit_r[pl.ds(v * LANES, LANES)]
                hp = hit_p[pl.ds(v * LANES, LANES)]
                m = lax.shift_right_arithmetic(
                    lax.shift_right_arithmetic(hr, 7) - lo, 4) == sub
                cnt_v = plsc.all_reduce_population_count(m)
                dest = o + plsc.cumsum(jnp.where(m, 1, 0)) - 1
                plsc.store_scatter(hit_r2, [dest], hr, mask=m)
                plsc.store_scatter(hit_p2, [dest], hp, mask=m)
                return o + cnt_v[0]

            off2 = lax.fori_loop(0, nv, rebin, off2)
        soff[16] = off2
        plsc.store_scatter(hit_r2, [off2 + iota16],
                           jnp.full((LANES,), -1, jnp.int32),
                           mask=iota16 < LANES)

        def fetch(p, buf, sem):
            off = pl.multiple_of(p * PANEL, PANEL)
            return pltpu.async_copy(tT.at[:, pl.ds(off, PANEL)], buf, sem)

        def drain_panel(buf, sem):
            pltpu.make_async_copy(
                tT.at[:, pl.ds(0, PANEL)], buf, sem).wait()

        def drain_rows(n):
            def w(_, c):
                pltpu.make_async_copy(
                    st_out.at[pl.ds(0, DIM)], rowbl.at[0], semr).wait()
                return c
            lax.fori_loop(0, n, w, 0)

        def extract(buf, hr, hp, pmi, vslot, width):
            for l in range(LANES):
                @pl.when(pmi[l] != 0)
                def _():
                    colv = jnp.broadcast_to(
                        jnp.bitwise_and(hr[l], PANEL - 1), (LANES,)
                    ).astype(jnp.int32)
                    slot = vslot * LANES + l
                    for j in range(DIM // LANES):
                        seg = plsc.load_gather(
                            buf, [iota16 + j * LANES, colv])
                        rowbl[slot, pl.ds(j * LANES, LANES)] = seg
                    dst = pl.multiple_of(hp[l] * DIM, DIM)
                    pltpu.async_copy(rowbl.at[slot],
                                     st_out.at[pl.ds(dst, DIM)], semr)

        def hits_for(p, buf, width):
            sub = lax.shift_right_arithmetic(p - lo, 4)
            o0 = soff[sub]
            o1 = soff[sub + 1]
            v0 = lax.shift_right_arithmetic(o0, 4)
            v1 = lax.shift_right_arithmetic(o1 + LANES - 1, 4)

            def hscan(v, c4):
                hr = hit_r2[pl.ds(v * LANES, LANES)]
                hp = hit_p2[pl.ds(v * LANES, LANES)]
                pm = lax.shift_right_arithmetic(hr, 7) == p
                pmi = jnp.where(pm, 1, 0)
                c = plsc.all_reduce_population_count(pm)[0]
                vslot = jnp.mod(v, 4)

                @pl.when(jnp.logical_and(vslot == 0, v > v0))
                def _():
                    drain_rows(c4)

                @pl.when(c > 0)
                def _():
                    extract(buf, hr, hp, pmi, vslot, width)
                return jnp.where(jnp.logical_and(vslot == 0, v > v0),
                                 c, c4 + c)

            c4f = lax.fori_loop(v0, v1, hscan, 0)
            drain_rows(c4f)

        rings = ((pb0, semp0), (pb1, semp1), (pb2, semp2))
        for b, (pb, semp) in enumerate(rings):
            @pl.when(lo + b < hi)
            def _(b=b, pb=pb, semp=semp):
                fetch(lo + b, pb, semp)

        nk3 = (hi_eff - lo + 3) // 3

        def panel_iter(k3, carry):
            for b, (pb, semp) in enumerate(rings):
                p = lo + 3 * k3 + b

                @pl.when(p < hi)
                def _(p=p, pb=pb, semp=semp):
                    drain_panel(pb, semp)
                    hits_for(p, pb, PANEL)

                    @pl.when(p + 3 < hi)
                    def _():
                        fetch(p + 3, pb, semp)

                @pl.when(jnp.logical_and(p == NFULL, is_last))
                def _(p=p):
                    hits_for(p, tb, DIM)
            return carry

        lax.fori_loop(0, nk3, panel_iter, 0)


def _dot_body(u_st, i_st, w64, b16, out,
              urows, irows, w_v, b_v, out_v, tr_v):
    wid = lax.axis_index("s") * _NC + lax.axis_index("c")
    base = wid * _BPW * DIM

    pltpu.sync_copy(u_st.at[pl.ds(base, _BPW * DIM)], urows)
    pltpu.sync_copy(i_st.at[pl.ds(base, _BPW * DIM)], irows)
    pltpu.sync_copy(w64, w_v)
    pltpu.sync_copy(b16, b_v)

    wvs = [w_v[pl.ds(j * LANES, LANES)] for j in range(DIM // LANES)]
    bv = b_v[...]
    scat_idx = lax.iota(jnp.int32, LANES) * LANES

    def group(g, carry):
        for l in range(LANES):
            b = g * LANES + l
            acc = jnp.zeros((LANES,), jnp.float32)
            for j in range(DIM // LANES):
                uv = urows[pl.ds(b * DIM + j * LANES, LANES)]
                iv = irows[pl.ds(b * DIM + j * LANES, LANES)]
                acc = acc + uv * iv * wvs[j]
            plsc.store_scatter(tr_v, [scat_idx + l], acc)
        tot = tr_v[pl.ds(0, LANES)]
        for l in range(1, LANES):
            tot = tot + tr_v[pl.ds(l * LANES, LANES)]
        x = tot + bv
        out_v[pl.ds(g * LANES, LANES)] = 1.0 / (1.0 + jnp.exp(-x))
        return carry

    lax.fori_loop(0, _NGROUP, group, 0)

    pltpu.sync_copy(out_v, out.at[pl.ds(wid * _BPW, _BPW)])


@jax.jit
def _gmf_sc(user, item, utT, itT, utail, itail, w64, b16):
    mesh = plsc.VectorSubcoreMesh(core_axis_name="c", subcore_axis_name="s")
    gather = functools.partial(
        pl.kernel,
        mesh=mesh,
        out_type=(jax.ShapeDtypeStruct((BATCH * DIM,), jnp.float32),
                  jax.ShapeDtypeStruct((BATCH * DIM,), jnp.float32)),
        scratch_types=[
            pltpu.VMEM((BATCH,), jnp.int32),
            pltpu.VMEM((_HCAP,), jnp.int32),
            pltpu.VMEM((_HCAP,), jnp.int32),
            pltpu.VMEM((_HCAP,), jnp.int32),
            pltpu.VMEM((_HCAP,), jnp.int32),
            pltpu.VMEM((DIM, PANEL), jnp.float32),
            pltpu.VMEM((DIM, PANEL), jnp.float32),
            pltpu.VMEM((DIM, PANEL), jnp.float32),
            pltpu.VMEM((DIM, DIM), jnp.float32),
            pltpu.VMEM((4 * LANES, DIM), jnp.float32),
            pltpu.SMEM((32,), jnp.int32),
            pltpu.SemaphoreType.DMA,
            pltpu.SemaphoreType.DMA,
            pltpu.SemaphoreType.DMA,
            pltpu.SemaphoreType.DMA,
        ],
        compiler_params=pltpu.CompilerParams(needs_layout_passes=False),
    )(_gather_body)
    u_st, i_st = gather(user, item, utT, itT, utail, itail)

    dot = functools.partial(
        pl.kernel,
        mesh=mesh,
        out_type=jax.ShapeDtypeStruct((BATCH,), jnp.float32),
        scratch_types=[
            pltpu.VMEM((_BPW * DIM,), jnp.float32),
            pltpu.VMEM((_BPW * DIM,), jnp.float32),
            pltpu.VMEM((DIM,), jnp.float32),
            pltpu.VMEM((LANES,), jnp.float32),
            pltpu.VMEM((_BPW,), jnp.float32),
            pltpu.VMEM((LANES * LANES,), jnp.float32),
        ],
        compiler_params=pltpu.CompilerParams(needs_layout_passes=False),
    )(_dot_body)
    return dot(u_st, i_st, w64, b16)


def kernel(user, item, user_table, item_table, dense_w, dense_b):
    utT = user_table.T
    itT = item_table.T
    utail = user_table[TAILS:TAILS + DIM].T
    itail = item_table[TAILS:TAILS + DIM].T
    w64 = dense_w.reshape(DIM)
    b16 = jnp.broadcast_to(dense_b, (LANES,))
    return _gmf_sc(user.astype(jnp.int32), item.astype(jnp.int32),
                   utT, itT, utail, itail, w64, b16)


# speedup vs baseline: 4.0013x; 1.2725x over previous
import functools

import jax
import jax.numpy as jnp
from jax import lax
from jax.experimental import pallas as pl
from jax.experimental.pallas import tpu as pltpu
from jax.experimental.pallas import tpu_sc as plsc

BATCH = 16384
DIM = 64
LANES = 16
PANEL = 128
NFULL = 7812
TAILS = NFULL * PANEL
ICHUNK = 2048

_info = plsc.get_sparse_core_info()
_NC, _NS = _info.num_cores, _info.num_subcores
_NW = _NC * _NS
_PPW = -(-NFULL // _NW)
_BPW = BATCH // _NW
_NGROUP = _BPW // LANES
_HCAP = BATCH + 2 * LANES


def _gather_body(user_h, item_h, utT, itT, utail, itail, u_out, i_out,
                 idx_v, hit_r, hit_p, hit_r2, hit_p2, pb0, pb1, pb2, pb3,
                 tb, rowbl, soff, semp0, semp1, semp2, semp3, semr):
    wid = lax.axis_index("s") * _NC + lax.axis_index("c")
    lo = wid * _PPW
    hi = jnp.minimum(lo + _PPW, NFULL)
    is_last = wid == (_NW - 1)
    hi_eff = hi + jnp.where(is_last, 1, 0)
    iota16 = lax.iota(jnp.int32, LANES)

    for idx_hbm, tT, tailT, st_out in ((user_h, utT, utail, u_out),
                                       (item_h, itT, itail, i_out)):
        pltpu.sync_copy(tailT, tb)

        hn = 0
        for ch in range(BATCH // ICHUNK):
            pltpu.sync_copy(idx_hbm.at[pl.ds(ch * ICHUNK, ICHUNK)], idx_v)

            def scan(v, off, ch=ch):
                rvec = idx_v[pl.ds(v * LANES, LANES)]
                pan = lax.shift_right_arithmetic(rvec, 7)
                m = jnp.logical_and(pan >= lo, pan < hi_eff)
                cnt_v = plsc.all_reduce_population_count(m)
                dest = off + plsc.cumsum(jnp.where(m, 1, 0)) - 1
                plsc.store_scatter(hit_r, [dest], rvec, mask=m)
                plsc.store_scatter(
                    hit_p, [dest],
                    ch * ICHUNK + v * LANES + iota16, mask=m)
                return off + cnt_v[0]

            hn = lax.fori_loop(0, ICHUNK // LANES, scan, hn)
        plsc.store_scatter(hit_r, [hn + iota16],
                           jnp.full((LANES,), -1, jnp.int32),
                           mask=iota16 < LANES)
        nv = lax.shift_right_arithmetic(hn + LANES - 1, 4)

        off2 = 0
        for sub in range(16):
            soff[sub] = off2

            def rebin(v, o, sub=sub):
                hr = hit_r[pl.ds(v * LANES, LANES)]
                hp = hit_p[pl.ds(v * LANES, LANES)]
                m = lax.shift_right_arithmetic(
                    lax.shift_right_arithmetic(hr, 7) - lo, 4) == sub
                cnt_v = plsc.all_reduce_population_count(m)
                dest = o + plsc.cumsum(jnp.where(m, 1, 0)) - 1
                plsc.store_scatter(hit_r2, [dest], hr, mask=m)
                plsc.store_scatter(hit_p2, [dest], hp, mask=m)
                return o + cnt_v[0]

            off2 = lax.fori_loop(0, nv, rebin, off2)
        soff[16] = off2
        plsc.store_scatter(hit_r2, [off2 + iota16],
                           jnp.full((LANES,), -1, jnp.int32),
                           mask=iota16 < LANES)

        def fetch(p, buf, sem):
            off = pl.multiple_of(p * PANEL, PANEL)
            return pltpu.async_copy(tT.at[:, pl.ds(off, PANEL)], buf, sem)

        def drain_panel(buf, sem):
            pltpu.make_async_copy(
                tT.at[:, pl.ds(0, PANEL)], buf, sem).wait()

        def drain_rows(n):
            def w(_, c):
                pltpu.make_async_copy(
                    st_out.at[pl.ds(0, DIM)], rowbl.at[0], semr).wait()
                return c
            lax.fori_loop(0, n, w, 0)

        def extract(buf, hr, hp, pmi, vslot, width):
            for l in range(LANES):
                @pl.when(pmi[l] != 0)
                def _():
                    colv = jnp.broadcast_to(
                        jnp.bitwise_and(hr[l], PANEL - 1), (LANES,)
                    ).astype(jnp.int32)
                    slot = vslot * LANES + l
                    for j in range(DIM // LANES):
                        seg = plsc.load_gather(
                            buf, [iota16 + j * LANES, colv])
                        rowbl[slot, pl.ds(j * LANES, LANES)] = seg
                    dst = pl.multiple_of(hp[l] * DIM, DIM)
                    pltpu.async_copy(rowbl.at[slot],
                                     st_out.at[pl.ds(dst, DIM)], semr)

        def hits_for(p, buf, width):
            sub = lax.shift_right_arithmetic(p - lo, 4)
            o0 = soff[sub]
            o1 = soff[sub + 1]
            v0 = lax.shift_right_arithmetic(o0, 4)
            v1 = lax.shift_right_arithmetic(o1 + LANES - 1, 4)

            def hscan(v, c4):
                hr = hit_r2[pl.ds(v * LANES, LANES)]
                hp = hit_p2[pl.ds(v * LANES, LANES)]
                pm = lax.shift_right_arithmetic(hr, 7) == p
                pmi = jnp.where(pm, 1, 0)
                c = plsc.all_reduce_population_count(pm)[0]
                vslot = jnp.mod(v, 4)

                @pl.when(jnp.logical_and(vslot == 0, v > v0))
                def _():
                    drain_rows(c4)

                @pl.when(c > 0)
                def _():
                    extract(buf, hr, hp, pmi, vslot, width)
                return jnp.where(jnp.logical_and(vslot == 0, v > v0),
                                 c, c4 + c)

            c4f = lax.fori_loop(v0, v1, hscan, 0)
            drain_rows(c4f)

        rings = ((pb0, semp0), (pb1, semp1), (pb2, semp2), (pb3, semp3))
        for b, (pb, semp) in enumerate(rings):
            @pl.when(lo + b < hi)
            def _(b=b, pb=pb, semp=semp):
                fetch(lo + b, pb, semp)

        nk4 = (hi - lo + 3) // 4

        def panel_iter(k4, carry):
            for b, (pb, semp) in enumerate(rings):
                p = lo + 4 * k4 + b

                @pl.when(p < hi)
                def _(p=p, pb=pb, semp=semp):
                    drain_panel(pb, semp)
                    hits_for(p, pb, PANEL)

                    @pl.when(p + 4 < hi)
                    def _():
                        fetch(p + 4, pb, semp)
            return carry

        lax.fori_loop(0, nk4, panel_iter, 0)

        @pl.when(is_last)
        def _():
            hits_for(NFULL, tb, DIM)


def _dot_body(u_st, i_st, w64, b16, out,
              urows, irows, w_v, b_v, out_v, tr_v):
    wid = lax.axis_index("s") * _NC + lax.axis_index("c")
    base = wid * _BPW * DIM

    pltpu.sync_copy(u_st.at[pl.ds(base, _BPW * DIM)], urows)
    pltpu.sync_copy(i_st.at[pl.ds(base, _BPW * DIM)], irows)
    pltpu.sync_copy(w64, w_v)
    pltpu.sync_copy(b16, b_v)

    wvs = [w_v[pl.ds(j * LANES, LANES)] for j in range(DIM // LANES)]
    bv = b_v[...]
    scat_idx = lax.iota(jnp.int32, LANES) * LANES

    def group(g, carry):
        for l in range(LANES):
            b = g * LANES + l
            acc = jnp.zeros((LANES,), jnp.float32)
            for j in range(DIM // LANES):
                uv = urows[pl.ds(b * DIM + j * LANES, LANES)]
                iv = irows[pl.ds(b * DIM + j * LANES, LANES)]
                acc = acc + uv * iv * wvs[j]
            plsc.store_scatter(tr_v, [scat_idx + l], acc)
        tot = tr_v[pl.ds(0, LANES)]
        for l in range(1, LANES):
            tot = tot + tr_v[pl.ds(l * LANES, LANES)]
        x = tot + bv
        out_v[pl.ds(g * LANES, LANES)] = 1.0 / (1.0 + jnp.exp(-x))
        return carry

    lax.fori_loop(0, _NGROUP, group, 0)

    pltpu.sync_copy(out_v, out.at[pl.ds(wid * _BPW, _BPW)])


@jax.jit
def _gmf_sc(user, item, utT, itT, utail, itail, w64, b16):
    mesh = plsc.VectorSubcoreMesh(core_axis_name="c", subcore_axis_name="s")
    gather = functools.partial(
        pl.kernel,
        mesh=mesh,
        out_type=(jax.ShapeDtypeStruct((BATCH * DIM,), jnp.float32),
                  jax.ShapeDtypeStruct((BATCH * DIM,), jnp.float32)),
        scratch_types=[
            pltpu.VMEM((ICHUNK,), jnp.int32),
            pltpu.VMEM((_HCAP,), jnp.int32),
            pltpu.VMEM((_HCAP,), jnp.int32),
            pltpu.VMEM((_HCAP,), jnp.int32),
            pltpu.VMEM((_HCAP,), jnp.int32),
            pltpu.VMEM((DIM, PANEL), jnp.float32),
            pltpu.VMEM((DIM, PANEL), jnp.float32),
            pltpu.VMEM((DIM, PANEL), jnp.float32),
            pltpu.VMEM((DIM, PANEL), jnp.float32),
            pltpu.VMEM((DIM, DIM), jnp.float32),
            pltpu.VMEM((4 * LANES, DIM), jnp.float32),
            pltpu.SMEM((32,), jnp.int32),
            pltpu.SemaphoreType.DMA,
            pltpu.SemaphoreType.DMA,
            pltpu.SemaphoreType.DMA,
            pltpu.SemaphoreType.DMA,
            pltpu.SemaphoreType.DMA,
        ],
        compiler_params=pltpu.CompilerParams(needs_layout_passes=False),
    )(_gather_body)
    u_st, i_st = gather(user, item, utT, itT, utail, itail)

    dot = functools.partial(
        pl.kernel,
        mesh=mesh,
        out_type=jax.ShapeDtypeStruct((BATCH,), jnp.float32),
        scratch_types=[
            pltpu.VMEM((_BPW * DIM,), jnp.float32),
            pltpu.VMEM((_BPW * DIM,), jnp.float32),
            pltpu.VMEM((DIM,), jnp.float32),
            pltpu.VMEM((LANES,), jnp.float32),
            pltpu.VMEM((_BPW,), jnp.float32),
            pltpu.VMEM((LANES * LANES,), jnp.float32),
        ],
        compiler_params=pltpu.CompilerParams(needs_layout_passes=False),
    )(_dot_body)
    return dot(u_st, i_st, w64, b16)


def kernel(user, item, user_table, item_table, dense_w, dense_b):
    utT = user_table.T
    itT = item_table.T
    utail = user_table[TAILS:TAILS + DIM].T
    itail = item_table[TAILS:TAILS + DIM].T
    w64 = dense_w.reshape(DIM)
    b16 = jnp.broadcast_to(dense_b, (LANES,))
    return _gmf_sc(user.astype(jnp.int32), item.astype(jnp.int32),
                   utT, itT, utail, itail, w64, b16)


# speedup vs baseline: 4.5504x; 1.1372x over previous
import functools

import jax
import jax.numpy as jnp
from jax import lax
from jax.experimental import pallas as pl
from jax.experimental.pallas import tpu as pltpu
from jax.experimental.pallas import tpu_sc as plsc

BATCH = 16384
DIM = 64
LANES = 16
PANEL = 256
PSHIFT = 8
SUBSHIFT = 3
NFULL = 3906
TAILS = NFULL * PANEL
ICHUNK = 2048

_info = plsc.get_sparse_core_info()
_NC, _NS = _info.num_cores, _info.num_subcores
_NW = _NC * _NS
_PPW = -(-NFULL // _NW)
_BPW = BATCH // _NW
_NGROUP = _BPW // LANES
_HCAP = BATCH + 2 * LANES


def _gather_body(user_h, item_h, utT, itT, utail, itail, u_out, i_out,
                 idx_v, hit_r, hit_p, hit_r2, hit_p2, pb0, pb1,
                 tb, rowbl, soff, semp0, semp1, semr):
    wid = lax.axis_index("s") * _NC + lax.axis_index("c")
    lo = wid * _PPW
    hi = jnp.minimum(lo + _PPW, NFULL)
    is_last = wid == (_NW - 1)
    hi_eff = hi + jnp.where(is_last, 1, 0)
    iota16 = lax.iota(jnp.int32, LANES)

    for idx_hbm, tT, tailT, st_out in ((user_h, utT, utail, u_out),
                                       (item_h, itT, itail, i_out)):
        pltpu.sync_copy(tailT, tb)

        hn = 0
        for ch in range(BATCH // ICHUNK):
            pltpu.sync_copy(idx_hbm.at[pl.ds(ch * ICHUNK, ICHUNK)], idx_v)

            def scan(v, off, ch=ch):
                rvec = idx_v[pl.ds(v * LANES, LANES)]
                pan = lax.shift_right_arithmetic(rvec, PSHIFT)
                m = jnp.logical_and(pan >= lo, pan < hi_eff)
                cnt_v = plsc.all_reduce_population_count(m)
                dest = off + plsc.cumsum(jnp.where(m, 1, 0)) - 1
                plsc.store_scatter(hit_r, [dest], rvec, mask=m)
                plsc.store_scatter(
                    hit_p, [dest],
                    ch * ICHUNK + v * LANES + iota16, mask=m)
                return off + cnt_v[0]

            hn = lax.fori_loop(0, ICHUNK // LANES, scan, hn)
        plsc.store_scatter(hit_r, [hn + iota16],
                           jnp.full((LANES,), -1, jnp.int32),
                           mask=iota16 < LANES)
        nv = lax.shift_right_arithmetic(hn + LANES - 1, 4)

        off2 = 0
        for sub in range(16):
            soff[sub] = off2

            def rebin(v, o, sub=sub):
                hr = hit_r[pl.ds(v * LANES, LANES)]
                hp = hit_p[pl.ds(v * LANES, LANES)]
                m = lax.shift_right_arithmetic(
                    lax.shift_right_arithmetic(hr, PSHIFT) - lo,
                    SUBSHIFT) == sub
                cnt_v = plsc.all_reduce_population_count(m)
                dest = o + plsc.cumsum(jnp.where(m, 1, 0)) - 1
                plsc.store_scatter(hit_r2, [dest], hr, mask=m)
                plsc.store_scatter(hit_p2, [dest], hp, mask=m)
                return o + cnt_v[0]

            off2 = lax.fori_loop(0, nv, rebin, off2)
        soff[16] = off2
        plsc.store_scatter(hit_r2, [off2 + iota16],
                           jnp.full((LANES,), -1, jnp.int32),
                           mask=iota16 < LANES)

        def fetch(p, buf, sem):
            off = pl.multiple_of(p * PANEL, PANEL)
            return pltpu.async_copy(tT.at[:, pl.ds(off, PANEL)], buf, sem)

        def drain_panel(buf, sem):
            pltpu.make_async_copy(
                tT.at[:, pl.ds(0, PANEL)], buf, sem).wait()

        def drain_rows(n):
            def w(_, c):
                pltpu.make_async_copy(
                    st_out.at[pl.ds(0, DIM)], rowbl.at[0], semr).wait()
                return c
            lax.fori_loop(0, n, w, 0)

        def extract(buf, hr, hp, pmi, vslot, width):
            for l in range(LANES):
                @pl.when(pmi[l] != 0)
                def _():
                    colv = jnp.broadcast_to(
                        jnp.bitwise_and(hr[l], PANEL - 1), (LANES,)
                    ).astype(jnp.int32)
                    slot = vslot * LANES + l
                    for j in range(DIM // LANES):
                        seg = plsc.load_gather(
                            buf, [iota16 + j * LANES, colv])
                        rowbl[slot, pl.ds(j * LANES, LANES)] = seg
                    dst = pl.multiple_of(hp[l] * DIM, DIM)
                    pltpu.async_copy(rowbl.at[slot],
                                     st_out.at[pl.ds(dst, DIM)], semr)

        def hits_for(p, buf, width):
            sub = lax.shift_right_arithmetic(p - lo, SUBSHIFT)
            o0 = soff[sub]
            o1 = soff[sub + 1]
            v0 = lax.shift_right_arithmetic(o0, 4)
            v1 = lax.shift_right_arithmetic(o1 + LANES - 1, 4)

            def hscan(v, c4):
                hr = hit_r2[pl.ds(v * LANES, LANES)]
                hp = hit_p2[pl.ds(v * LANES, LANES)]
                pm = lax.shift_right_arithmetic(hr, PSHIFT) == p
                pmi = jnp.where(pm, 1, 0)
                c = plsc.all_reduce_population_count(pm)[0]
                vslot = jnp.mod(v, 4)

                @pl.when(jnp.logical_and(vslot == 0, v > v0))
                def _():
                    drain_rows(c4)

                @pl.when(c > 0)
                def _():
                    extract(buf, hr, hp, pmi, vslot, width)
                return jnp.where(jnp.logical_and(vslot == 0, v > v0),
                                 c, c4 + c)

            c4f = lax.fori_loop(v0, v1, hscan, 0)
            drain_rows(c4f)

        rings = ((pb0, semp0), (pb1, semp1))
        for b, (pb, semp) in enumerate(rings):
            @pl.when(lo + b < hi)
            def _(b=b, pb=pb, semp=semp):
                fetch(lo + b, pb, semp)

        nk2 = (hi - lo + 1) // 2

        def panel_iter(k2, carry):
            for b, (pb, semp) in enumerate(rings):
                p = lo + 2 * k2 + b

                @pl.when(p < hi)
                def _(p=p, pb=pb, semp=semp):
                    drain_panel(pb, semp)
                    hits_for(p, pb, PANEL)

                    @pl.when(p + 2 < hi)
                    def _():
                        fetch(p + 2, pb, semp)
            return carry

        lax.fori_loop(0, nk2, panel_iter, 0)

        @pl.when(is_last)
        def _():
            hits_for(NFULL, tb, DIM)


def _dot_body(u_st, i_st, w64, b16, out,
              urows, irows, w_v, b_v, out_v, tr_v):
    wid = lax.axis_index("s") * _NC + lax.axis_index("c")
    base = wid * _BPW * DIM

    pltpu.sync_copy(u_st.at[pl.ds(base, _BPW * DIM)], urows)
    pltpu.sync_copy(i_st.at[pl.ds(base, _BPW * DIM)], irows)
    pltpu.sync_copy(w64, w_v)
    pltpu.sync_copy(b16, b_v)

    wvs = [w_v[pl.ds(j * LANES, LANES)] for j in range(DIM // LANES)]
    bv = b_v[...]
    scat_idx = lax.iota(jnp.int32, LANES) * LANES

    def group(g, carry):
        for l in range(LANES):
            b = g * LANES + l
            acc = jnp.zeros((LANES,), jnp.float32)
            for j in range(DIM // LANES):
                uv = urows[pl.ds(b * DIM + j * LANES, LANES)]
                iv = irows[pl.ds(b * DIM + j * LANES, LANES)]
                acc = acc + uv * iv * wvs[j]
            plsc.store_scatter(tr_v, [scat_idx + l], acc)
        tot = tr_v[pl.ds(0, LANES)]
        for l in range(1, LANES):
            tot = tot + tr_v[pl.ds(l * LANES, LANES)]
        x = tot + bv
        out_v[pl.ds(g * LANES, LANES)] = 1.0 / (1.0 + jnp.exp(-x))
        return carry

    lax.fori_loop(0, _NGROUP, group, 0)

    pltpu.sync_copy(out_v, out.at[pl.ds(wid * _BPW, _BPW)])


@jax.jit
def _gmf_sc(user, item, utT, itT, utail, itail, w64, b16):
    mesh = plsc.VectorSubcoreMesh(core_axis_name="c", subcore_axis_name="s")
    gather = functools.partial(
        pl.kernel,
        mesh=mesh,
        out_type=(jax.ShapeDtypeStruct((BATCH * DIM,), jnp.float32),
                  jax.ShapeDtypeStruct((BATCH * DIM,), jnp.float32)),
        scratch_types=[
            pltpu.VMEM((ICHUNK,), jnp.int32),
            pltpu.VMEM((_HCAP,), jnp.int32),
            pltpu.VMEM((_HCAP,), jnp.int32),
            pltpu.VMEM((_HCAP,), jnp.int32),
            pltpu.VMEM((_HCAP,), jnp.int32),
            pltpu.VMEM((DIM, PANEL), jnp.float32),
            pltpu.VMEM((DIM, PANEL), jnp.float32),
            pltpu.VMEM((DIM, DIM), jnp.float32),
            pltpu.VMEM((4 * LANES, DIM), jnp.float32),
            pltpu.SMEM((32,), jnp.int32),
            pltpu.SemaphoreType.DMA,
            pltpu.SemaphoreType.DMA,
            pltpu.SemaphoreType.DMA,
        ],
        compiler_params=pltpu.CompilerParams(needs_layout_passes=False),
    )(_gather_body)
    u_st, i_st = gather(user, item, utT, itT, utail, itail)

    dot = functools.partial(
        pl.kernel,
        mesh=mesh,
        out_type=jax.ShapeDtypeStruct((BATCH,), jnp.float32),
        scratch_types=[
            pltpu.VMEM((_BPW * DIM,), jnp.float32),
            pltpu.VMEM((_BPW * DIM,), jnp.float32),
            pltpu.VMEM((DIM,), jnp.float32),
            pltpu.VMEM((LANES,), jnp.float32),
            pltpu.VMEM((_BPW,), jnp.float32),
            pltpu.VMEM((LANES * LANES,), jnp.float32),
        ],
        compiler_params=pltpu.CompilerParams(needs_layout_passes=False),
    )(_dot_body)
    return dot(u_st, i_st, w64, b16)


def kernel(user, item, user_table, item_table, dense_w, dense_b):
    utT = user_table.T
    itT = item_table.T
    utail = user_table[TAILS:TAILS + DIM].T
    itail = item_table[TAILS:TAILS + DIM].T
    w64 = dense_w.reshape(DIM)
    b16 = jnp.broadcast_to(dense_b, (LANES,))
    return _gmf_sc(user.astype(jnp.int32), item.astype(jnp.int32),
                   utT, itT, utail, itail, w64, b16)


# speedup vs baseline: 4.5946x; 1.0097x over previous
import functools

import jax
import jax.numpy as jnp
from jax import lax
from jax.experimental import pallas as pl
from jax.experimental.pallas import tpu as pltpu
from jax.experimental.pallas import tpu_sc as plsc

BATCH = 16384
DIM = 64
LANES = 16
PANEL = 256
PSHIFT = 8
SUBSHIFT = 3
NFULL = 3906
TAILS = NFULL * PANEL
ICHUNK = 2048

_info = plsc.get_sparse_core_info()
_NC, _NS = _info.num_cores, _info.num_subcores
_NW = _NC * _NS
_PPW = -(-NFULL // _NW)
_BPW = BATCH // _NW
_NGROUP = _BPW // LANES
_HCAP = BATCH + 2 * LANES


def _gather_body(user_h, item_h, utT, itT, utail, itail, u_out, i_out,
                 idx_v, hit_p, hit_p2, pb0, pb1, pb2,
                 tb, rowbl, soff, semp0, semp1, semp2, semr):
    wid = lax.axis_index("s") * _NC + lax.axis_index("c")
    lo = wid * _PPW
    hi = jnp.minimum(lo + _PPW, NFULL)
    is_last = wid == (_NW - 1)
    hi_eff = hi + jnp.where(is_last, 1, 0)
    iota16 = lax.iota(jnp.int32, LANES)

    for idx_hbm, tT, tailT, st_out in ((user_h, utT, utail, u_out),
                                       (item_h, itT, itail, i_out)):
        pltpu.sync_copy(tailT, tb)
        pltpu.sync_copy(idx_hbm, idx_v.at[pl.ds(0, BATCH)])
        idx_v[pl.ds(BATCH, LANES)] = jnp.full((LANES,), -1, jnp.int32)

        def scan(v, off):
            rvec = idx_v[pl.ds(v * LANES, LANES)]
            pan = lax.shift_right_arithmetic(rvec, PSHIFT)
            m = jnp.logical_and(pan >= lo, pan < hi_eff)
            cnt_v = plsc.all_reduce_population_count(m)
            dest = off + plsc.cumsum(jnp.where(m, 1, 0)) - 1
            plsc.store_scatter(hit_p, [dest], v * LANES + iota16, mask=m)
            return off + cnt_v[0]

        hn = lax.fori_loop(0, BATCH // LANES, scan, 0)
        plsc.store_scatter(hit_p, [hn + iota16],
                           jnp.full((LANES,), BATCH, jnp.int32),
                           mask=iota16 < LANES)
        nv = lax.shift_right_arithmetic(hn + LANES - 1, 4)

        off2 = 0
        for sub in range(16):
            soff[sub] = off2

            def rebin(v, o, sub=sub):
                hp = hit_p[pl.ds(v * LANES, LANES)]
                hr = plsc.load_gather(idx_v, [hp])
                m = lax.shift_right_arithmetic(
                    lax.shift_right_arithmetic(hr, PSHIFT) - lo,
                    SUBSHIFT) == sub
                cnt_v = plsc.all_reduce_population_count(m)
                dest = o + plsc.cumsum(jnp.where(m, 1, 0)) - 1
                plsc.store_scatter(hit_p2, [dest], hp, mask=m)
                return o + cnt_v[0]

            off2 = lax.fori_loop(0, nv, rebin, off2)
        soff[16] = off2
        plsc.store_scatter(hit_p2, [off2 + iota16],
                           jnp.full((LANES,), BATCH, jnp.int32),
                           mask=iota16 < LANES)

        def fetch(p, buf, sem):
            off = pl.multiple_of(p * PANEL, PANEL)
            return pltpu.async_copy(tT.at[:, pl.ds(off, PANEL)], buf, sem)

        def drain_panel(buf, sem):
            pltpu.make_async_copy(
                tT.at[:, pl.ds(0, PANEL)], buf, sem).wait()

        def drain_rows(n):
            def w(_, c):
                pltpu.make_async_copy(
                    st_out.at[pl.ds(0, DIM)], rowbl.at[0], semr).wait()
                return c
            lax.fori_loop(0, n, w, 0)

        def extract(buf, hr, hp, pmi, vslot, width):
            for l in range(LANES):
                @pl.when(pmi[l] != 0)
                def _():
                    colv = jnp.broadcast_to(
                        jnp.bitwise_and(hr[l], PANEL - 1), (LANES,)
                    ).astype(jnp.int32)
                    slot = vslot * LANES + l
                    for j in range(DIM // LANES):
                        seg = plsc.load_gather(
                            buf, [iota16 + j * LANES, colv])
                        rowbl[slot, pl.ds(j * LANES, LANES)] = seg
                    dst = pl.multiple_of(hp[l] * DIM, DIM)
                    pltpu.async_copy(rowbl.at[slot],
                                     st_out.at[pl.ds(dst, DIM)], semr)

        def hits_for(p, buf, width):
            sub = lax.shift_right_arithmetic(p - lo, SUBSHIFT)
            o0 = soff[sub]
            o1 = soff[sub + 1]
            v0 = lax.shift_right_arithmetic(o0, 4)
            v1 = lax.shift_right_arithmetic(o1 + LANES - 1, 4)

            def hscan(v, c4):
                hp = hit_p2[pl.ds(v * LANES, LANES)]
                hr = plsc.load_gather(idx_v, [hp])
                pm = lax.shift_right_arithmetic(hr, PSHIFT) == p
                pmi = jnp.where(pm, 1, 0)
                c = plsc.all_reduce_population_count(pm)[0]
                vslot = jnp.mod(v, 4)

                @pl.when(jnp.logical_and(vslot == 0, v > v0))
                def _():
                    drain_rows(c4)

                @pl.when(c > 0)
                def _():
                    extract(buf, hr, hp, pmi, vslot, width)
                return jnp.where(jnp.logical_and(vslot == 0, v > v0),
                                 c, c4 + c)

            c4f = lax.fori_loop(v0, v1, hscan, 0)
            drain_rows(c4f)

        rings = ((pb0, semp0), (pb1, semp1), (pb2, semp2))
        for b, (pb, semp) in enumerate(rings):
            @pl.when(lo + b < hi)
            def _(b=b, pb=pb, semp=semp):
                fetch(lo + b, pb, semp)

        nk3 = (hi - lo + 2) // 3

        def panel_iter(k3, carry):
            for b, (pb, semp) in enumerate(rings):
                p = lo + 3 * k3 + b

                @pl.when(p < hi)
                def _(p=p, pb=pb, semp=semp):
                    drain_panel(pb, semp)
                    hits_for(p, pb, PANEL)

                    @pl.when(p + 3 < hi)
                    def _():
                        fetch(p + 3, pb, semp)
            return carry

        lax.fori_loop(0, nk3, panel_iter, 0)

        @pl.when(is_last)
        def _():
            hits_for(NFULL, tb, DIM)


def _dot_body(u_st, i_st, w64, b16, out,
              urows, irows, w_v, b_v, out_v, tr_v):
    wid = lax.axis_index("s") * _NC + lax.axis_index("c")
    base = wid * _BPW * DIM

    pltpu.sync_copy(u_st.at[pl.ds(base, _BPW * DIM)], urows)
    pltpu.sync_copy(i_st.at[pl.ds(base, _BPW * DIM)], irows)
    pltpu.sync_copy(w64, w_v)
    pltpu.sync_copy(b16, b_v)

    wvs = [w_v[pl.ds(j * LANES, LANES)] for j in range(DIM // LANES)]
    bv = b_v[...]
    scat_idx = lax.iota(jnp.int32, LANES) * LANES

    def group(g, carry):
        for l in range(LANES):
            b = g * LANES + l
            acc = jnp.zeros((LANES,), jnp.float32)
            for j in range(DIM // LANES):
                uv = urows[pl.ds(b * DIM + j * LANES, LANES)]
                iv = irows[pl.ds(b * DIM + j * LANES, LANES)]
                acc = acc + uv * iv * wvs[j]
            plsc.store_scatter(tr_v, [scat_idx + l], acc)
        tot = tr_v[pl.ds(0, LANES)]
        for l in range(1, LANES):
            tot = tot + tr_v[pl.ds(l * LANES, LANES)]
        x = tot + bv
        out_v[pl.ds(g * LANES, LANES)] = 1.0 / (1.0 + jnp.exp(-x))
        return carry

    lax.fori_loop(0, _NGROUP, group, 0)

    pltpu.sync_copy(out_v, out.at[pl.ds(wid * _BPW, _BPW)])


@jax.jit
def _gmf_sc(user, item, utT, itT, utail, itail, w64, b16):
    mesh = plsc.VectorSubcoreMesh(core_axis_name="c", subcore_axis_name="s")
    gather = functools.partial(
        pl.kernel,
        mesh=mesh,
        out_type=(jax.ShapeDtypeStruct((BATCH * DIM,), jnp.float32),
                  jax.ShapeDtypeStruct((BATCH * DIM,), jnp.float32)),
        scratch_types=[
            pltpu.VMEM((BATCH + LANES,), jnp.int32),
            pltpu.VMEM((_HCAP,), jnp.int32),
            pltpu.VMEM((_HCAP,), jnp.int32),
            pltpu.VMEM((DIM, PANEL), jnp.float32),
            pltpu.VMEM((DIM, PANEL), jnp.float32),
            pltpu.VMEM((DIM, PANEL), jnp.float32),
            pltpu.VMEM((DIM, DIM), jnp.float32),
            pltpu.VMEM((4 * LANES, DIM), jnp.float32),
            pltpu.SMEM((32,), jnp.int32),
            pltpu.SemaphoreType.DMA,
            pltpu.SemaphoreType.DMA,
            pltpu.SemaphoreType.DMA,
            pltpu.SemaphoreType.DMA,
        ],
        compiler_params=pltpu.CompilerParams(needs_layout_passes=False),
    )(_gather_body)
    u_st, i_st = gather(user, item, utT, itT, utail, itail)

    dot = functools.partial(
        pl.kernel,
        mesh=mesh,
        out_type=jax.ShapeDtypeStruct((BATCH,), jnp.float32),
        scratch_types=[
            pltpu.VMEM((_BPW * DIM,), jnp.float32),
            pltpu.VMEM((_BPW * DIM,), jnp.float32),
            pltpu.VMEM((DIM,), jnp.float32),
            pltpu.VMEM((LANES,), jnp.float32),
            pltpu.VMEM((_BPW,), jnp.float32),
            pltpu.VMEM((LANES * LANES,), jnp.float32),
        ],
        compiler_params=pltpu.CompilerParams(needs_layout_passes=False),
    )(_dot_body)
    return dot(u_st, i_st, w64, b16)


def kernel(user, item, user_table, item_table, dense_w, dense_b):
    utT = user_table.T
    itT = item_table.T
    utail = user_table[TAILS:TAILS + DIM].T
    itail = item_table[TAILS:TAILS + DIM].T
    w64 = dense_w.reshape(DIM)
    b16 = jnp.broadcast_to(dense_b, (LANES,))
    return _gmf_sc(user.astype(jnp.int32), item.astype(jnp.int32),
                   utT, itT, utail, itail, w64, b16)


# speedup vs baseline: 6.0566x; 1.3182x over previous
import functools

import jax
import jax.numpy as jnp
from jax import lax
from jax.experimental import pallas as pl
from jax.experimental.pallas import tpu as pltpu
from jax.experimental.pallas import tpu_sc as plsc

BATCH = 16384
DIM = 64
LANES = 16
PANEL = 512
PSHIFT = 9
SUBSHIFT = 2
NFULL = 1953
TAILS = NFULL * PANEL
ICHUNK = 2048

_info = plsc.get_sparse_core_info()
_NC, _NS = _info.num_cores, _info.num_subcores
_NW = _NC * _NS
_PPW = -(-NFULL // _NW)
_BPW = BATCH // _NW
_NGROUP = _BPW // LANES
_HCAP = BATCH + 2 * LANES


def _gather_body(user_h, item_h, utT, itT, utail, itail, u_out, i_out,
                 idx_v, hit_p, hit_p2, pb0, pb1,
                 tb, rowbl, soff, semp0, semp1, semr):
    wid = lax.axis_index("s") * _NC + lax.axis_index("c")
    lo = wid * _PPW
    hi = jnp.minimum(lo + _PPW, NFULL)
    is_last = wid == (_NW - 1)
    hi_eff = hi + jnp.where(is_last, 1, 0)
    iota16 = lax.iota(jnp.int32, LANES)

    for idx_hbm, tT, tailT, st_out in ((user_h, utT, utail, u_out),
                                       (item_h, itT, itail, i_out)):
        pltpu.sync_copy(tailT, tb)
        pltpu.sync_copy(idx_hbm, idx_v.at[pl.ds(0, BATCH)])
        idx_v[pl.ds(BATCH, LANES)] = jnp.full((LANES,), -1, jnp.int32)

        def scan(v, off):
            rvec = idx_v[pl.ds(v * LANES, LANES)]
            pan = lax.shift_right_arithmetic(rvec, PSHIFT)
            m = jnp.logical_and(pan >= lo, pan < hi_eff)
            cnt_v = plsc.all_reduce_population_count(m)
            dest = off + plsc.cumsum(jnp.where(m, 1, 0)) - 1
            plsc.store_scatter(hit_p, [dest], v * LANES + iota16, mask=m)
            return off + cnt_v[0]

        hn = lax.fori_loop(0, BATCH // LANES, scan, 0)
        plsc.store_scatter(hit_p, [hn + iota16],
                           jnp.full((LANES,), BATCH, jnp.int32),
                           mask=iota16 < LANES)
        nv = lax.shift_right_arithmetic(hn + LANES - 1, 4)

        off2 = 0
        for sub in range(16):
            soff[sub] = off2

            def rebin(v, o, sub=sub):
                hp = hit_p[pl.ds(v * LANES, LANES)]
                hr = plsc.load_gather(idx_v, [hp])
                m = lax.shift_right_arithmetic(
                    lax.shift_right_arithmetic(hr, PSHIFT) - lo,
                    SUBSHIFT) == sub
                cnt_v = plsc.all_reduce_population_count(m)
                dest = o + plsc.cumsum(jnp.where(m, 1, 0)) - 1
                plsc.store_scatter(hit_p2, [dest], hp, mask=m)
                return o + cnt_v[0]

            off2 = lax.fori_loop(0, nv, rebin, off2)
        soff[16] = off2
        plsc.store_scatter(hit_p2, [off2 + iota16],
                           jnp.full((LANES,), BATCH, jnp.int32),
                           mask=iota16 < LANES)

        def fetch(p, buf, sem):
            off = pl.multiple_of(p * PANEL, PANEL)
            return pltpu.async_copy(tT.at[:, pl.ds(off, PANEL)], buf, sem)

        def drain_panel(buf, sem):
            pltpu.make_async_copy(
                tT.at[:, pl.ds(0, PANEL)], buf, sem).wait()

        def drain_rows(n):
            def w(_, c):
                pltpu.make_async_copy(
                    st_out.at[pl.ds(0, DIM)], rowbl.at[0], semr).wait()
                return c
            lax.fori_loop(0, n, w, 0)

        def extract(buf, hr, hp, pmi, vslot, width):
            for l in range(LANES):
                @pl.when(pmi[l] != 0)
                def _():
                    colv = jnp.broadcast_to(
                        jnp.bitwise_and(hr[l], PANEL - 1), (LANES,)
                    ).astype(jnp.int32)
                    slot = vslot * LANES + l
                    for j in range(DIM // LANES):
                        seg = plsc.load_gather(
                            buf, [iota16 + j * LANES, colv])
                        rowbl[slot, pl.ds(j * LANES, LANES)] = seg
                    dst = pl.multiple_of(hp[l] * DIM, DIM)
                    pltpu.async_copy(rowbl.at[slot],
                                     st_out.at[pl.ds(dst, DIM)], semr)

        def hits_for(p, buf, width):
            sub = lax.shift_right_arithmetic(p - lo, SUBSHIFT)
            o0 = soff[sub]
            o1 = soff[sub + 1]
            v0 = lax.shift_right_arithmetic(o0, 4)
            v1 = lax.shift_right_arithmetic(o1 + LANES - 1, 4)

            def hscan(v, c4):
                hp = hit_p2[pl.ds(v * LANES, LANES)]
                hr = plsc.load_gather(idx_v, [hp])
                pm = lax.shift_right_arithmetic(hr, PSHIFT) == p
                pmi = jnp.where(pm, 1, 0)
                c = plsc.all_reduce_population_count(pm)[0]
                vslot = jnp.mod(v, 2)

                @pl.when(jnp.logical_and(vslot == 0, v > v0))
                def _():
                    drain_rows(c4)

                @pl.when(c > 0)
                def _():
                    extract(buf, hr, hp, pmi, vslot, width)
                return jnp.where(jnp.logical_and(vslot == 0, v > v0),
                                 c, c4 + c)

            c4f = lax.fori_loop(v0, v1, hscan, 0)
            drain_rows(c4f)

        rings = ((pb0, semp0), (pb1, semp1))
        for b, (pb, semp) in enumerate(rings):
            @pl.when(lo + b < hi)
            def _(b=b, pb=pb, semp=semp):
                fetch(lo + b, pb, semp)

        nk2 = (hi - lo + 1) // 2

        def panel_iter(k2, carry):
            for b, (pb, semp) in enumerate(rings):
                p = lo + 2 * k2 + b

                @pl.when(p < hi)
                def _(p=p, pb=pb, semp=semp):
                    drain_panel(pb, semp)
                    hits_for(p, pb, PANEL)

                    @pl.when(p + 2 < hi)
                    def _():
                        fetch(p + 2, pb, semp)
            return carry

        lax.fori_loop(0, nk2, panel_iter, 0)

        @pl.when(is_last)
        def _():
            hits_for(NFULL, tb, DIM)


def _dot_body(u_st, i_st, w64, b16, out,
              urows, irows, w_v, b_v, out_v, tr_v):
    wid = lax.axis_index("s") * _NC + lax.axis_index("c")
    base = wid * _BPW * DIM

    pltpu.sync_copy(u_st.at[pl.ds(base, _BPW * DIM)], urows)
    pltpu.sync_copy(i_st.at[pl.ds(base, _BPW * DIM)], irows)
    pltpu.sync_copy(w64, w_v)
    pltpu.sync_copy(b16, b_v)

    wvs = [w_v[pl.ds(j * LANES, LANES)] for j in range(DIM // LANES)]
    bv = b_v[...]
    scat_idx = lax.iota(jnp.int32, LANES) * LANES

    def group(g, carry):
        for l in range(LANES):
            b = g * LANES + l
            acc = jnp.zeros((LANES,), jnp.float32)
            for j in range(DIM // LANES):
                uv = urows[pl.ds(b * DIM + j * LANES, LANES)]
                iv = irows[pl.ds(b * DIM + j * LANES, LANES)]
                acc = acc + uv * iv * wvs[j]
            plsc.store_scatter(tr_v, [scat_idx + l], acc)
        tot = tr_v[pl.ds(0, LANES)]
        for l in range(1, LANES):
            tot = tot + tr_v[pl.ds(l * LANES, LANES)]
        x = tot + bv
        out_v[pl.ds(g * LANES, LANES)] = 1.0 / (1.0 + jnp.exp(-x))
        return carry

    lax.fori_loop(0, _NGROUP, group, 0)

    pltpu.sync_copy(out_v, out.at[pl.ds(wid * _BPW, _BPW)])


@jax.jit
def _gmf_sc(user, item, utT, itT, utail, itail, w64, b16):
    mesh = plsc.VectorSubcoreMesh(core_axis_name="c", subcore_axis_name="s")
    gather = functools.partial(
        pl.kernel,
        mesh=mesh,
        out_type=(jax.ShapeDtypeStruct((BATCH * DIM,), jnp.float32),
                  jax.ShapeDtypeStruct((BATCH * DIM,), jnp.float32)),
        scratch_types=[
            pltpu.VMEM((BATCH + LANES,), jnp.int32),
            pltpu.VMEM((_HCAP,), jnp.int32),
            pltpu.VMEM((_HCAP,), jnp.int32),
            pltpu.VMEM((DIM, PANEL), jnp.float32),
            pltpu.VMEM((DIM, PANEL), jnp.float32),
            pltpu.VMEM((DIM, DIM), jnp.float32),
            pltpu.VMEM((2 * LANES, DIM), jnp.float32),
            pltpu.SMEM((32,), jnp.int32),
            pltpu.SemaphoreType.DMA,
            pltpu.SemaphoreType.DMA,
            pltpu.SemaphoreType.DMA,
        ],
        compiler_params=pltpu.CompilerParams(needs_layout_passes=False),
    )(_gather_body)
    u_st, i_st = gather(user, item, utT, itT, utail, itail)

    dot = functools.partial(
        pl.kernel,
        mesh=mesh,
        out_type=jax.ShapeDtypeStruct((BATCH,), jnp.float32),
        scratch_types=[
            pltpu.VMEM((_BPW * DIM,), jnp.float32),
            pltpu.VMEM((_BPW * DIM,), jnp.float32),
            pltpu.VMEM((DIM,), jnp.float32),
            pltpu.VMEM((LANES,), jnp.float32),
            pltpu.VMEM((_BPW,), jnp.float32),
            pltpu.VMEM((LANES * LANES,), jnp.float32),
        ],
        compiler_params=pltpu.CompilerParams(needs_layout_passes=False),
    )(_dot_body)
    return dot(u_st, i_st, w64, b16)


def kernel(user, item, user_table, item_table, dense_w, dense_b):
    utT = user_table.T
    itT = item_table.T
    utail = user_table[TAILS:TAILS + DIM].T
    itail = item_table[TAILS:TAILS + DIM].T
    w64 = dense_w.reshape(DIM)
    b16 = jnp.broadcast_to(dense_b, (LANES,))
    return _gmf_sc(user.astype(jnp.int32), item.astype(jnp.int32),
                   utT, itT, utail, itail, w64, b16)


# speedup vs baseline: 6.0607x; 1.0007x over previous
import functools

import jax
import jax.numpy as jnp
from jax import lax
from jax.experimental import pallas as pl
from jax.experimental.pallas import tpu as pltpu
from jax.experimental.pallas import tpu_sc as plsc

BATCH = 16384
DIM = 64
LANES = 16
PANEL = 512
PSHIFT = 9
SUBSHIFT = 2
NFULL = 1953
TAILS = NFULL * PANEL

_info = plsc.get_sparse_core_info()
_NC, _NS = _info.num_cores, _info.num_subcores
_NW = _NC * _NS
_PPW = -(-NFULL // _NW)
_BPW = BATCH // _NW
_NGROUP = _BPW // LANES
_HCAP = BATCH + 2 * LANES


def _gather_body(user_h, item_h, utT, itT, utail, itail, u_out, i_out,
                 idx_v, hit_p, hit_p2, pb0, pb1,
                 tb, rowbl, soff, semp0, semp1, semr):
    wid = lax.axis_index("s") * _NC + lax.axis_index("c")
    lo = wid * _PPW
    hi = jnp.minimum(lo + _PPW, NFULL)
    is_last = wid == (_NW - 1)
    hi_eff = hi + jnp.where(is_last, 1, 0)
    iota16 = lax.iota(jnp.int32, LANES)

    for idx_hbm, tT, tailT, st_out in ((user_h, utT, utail, u_out),
                                       (item_h, itT, itail, i_out)):
        pltpu.sync_copy(tailT, tb)
        pltpu.sync_copy(idx_hbm, idx_v.at[pl.ds(0, BATCH)])
        idx_v[pl.ds(BATCH, LANES)] = jnp.full((LANES,), -1, jnp.int32)

        def scan(v, off):
            rvec = idx_v[pl.ds(v * LANES, LANES)]
            pan = lax.shift_right_arithmetic(rvec, PSHIFT)
            m = jnp.logical_and(pan >= lo, pan < hi_eff)
            cnt_v = plsc.all_reduce_population_count(m)
            dest = off + plsc.cumsum(jnp.where(m, 1, 0)) - 1
            plsc.store_scatter(hit_p, [dest], v * LANES + iota16, mask=m)
            return off + cnt_v[0]

        hn = lax.fori_loop(0, BATCH // LANES, scan, 0)
        plsc.store_scatter(hit_p, [hn + iota16],
                           jnp.full((LANES,), BATCH, jnp.int32),
                           mask=iota16 < LANES)
        nv = lax.shift_right_arithmetic(hn + LANES - 1, 4)

        off2 = 0
        for sub in range(16):
            soff[sub] = off2

            def rebin(v, o, sub=sub):
                hp = hit_p[pl.ds(v * LANES, LANES)]
                hr = plsc.load_gather(idx_v, [hp])
                m = lax.shift_right_arithmetic(
                    lax.shift_right_arithmetic(hr, PSHIFT) - lo,
                    SUBSHIFT) == sub
                cnt_v = plsc.all_reduce_population_count(m)
                dest = o + plsc.cumsum(jnp.where(m, 1, 0)) - 1
                plsc.store_scatter(hit_p2, [dest], hp, mask=m)
                return o + cnt_v[0]

            off2 = lax.fori_loop(0, nv, rebin, off2)
        soff[16] = off2
        plsc.store_scatter(hit_p2, [off2 + iota16],
                           jnp.full((LANES,), BATCH, jnp.int32),
                           mask=iota16 < LANES)

        def fetch(p, buf, sem):
            off = pl.multiple_of(p * PANEL, PANEL)
            return pltpu.async_copy(tT.at[:, pl.ds(off, PANEL)], buf, sem)

        def drain_panel(buf, sem):
            pltpu.make_async_copy(
                tT.at[:, pl.ds(0, PANEL)], buf, sem).wait()

        def drain_rows(n):
            def w(_, c):
                pltpu.make_async_copy(
                    st_out.at[pl.ds(0, DIM)], rowbl.at[0], semr).wait()
                return c
            lax.fori_loop(0, n, w, 0)

        def extract(buf, hr, hp, pmi, vslot, width):
            for l in range(LANES):
                @pl.when(pmi[l] != 0)
                def _():
                    colv = jnp.broadcast_to(
                        jnp.bitwise_and(hr[l], PANEL - 1), (LANES,)
                    ).astype(jnp.int32)
                    slot = vslot * LANES + l
                    for j in range(DIM // LANES):
                        seg = plsc.load_gather(
                            buf, [iota16 + j * LANES, colv])
                        rowbl[slot, pl.ds(j * LANES, LANES)] = seg
                    dst = pl.multiple_of(hp[l] * DIM, DIM)
                    pltpu.async_copy(rowbl.at[slot],
                                     st_out.at[pl.ds(dst, DIM)], semr)

        def hits_for(p, buf, width):
            sub = lax.shift_right_arithmetic(p - lo, SUBSHIFT)
            o0 = soff[sub]
            o1 = soff[sub + 1]
            v0 = lax.shift_right_arithmetic(o0, 4)
            v1 = lax.shift_right_arithmetic(o1 + LANES - 1, 4)

            def hscan(v, c4):
                hp = hit_p2[pl.ds(v * LANES, LANES)]
                hr = plsc.load_gather(idx_v, [hp])
                pm = lax.shift_right_arithmetic(hr, PSHIFT) == p
                pmi = jnp.where(pm, 1, 0)
                c = plsc.all_reduce_population_count(pm)[0]
                vslot = jnp.mod(v, 2)

                @pl.when(jnp.logical_and(vslot == 0, v > v0))
                def _():
                    drain_rows(c4)

                @pl.when(c > 0)
                def _():
                    extract(buf, hr, hp, pmi, vslot, width)
                return jnp.where(jnp.logical_and(vslot == 0, v > v0),
                                 c, c4 + c)

            c4f = lax.fori_loop(v0, v1, hscan, 0)
            drain_rows(c4f)

        rings = ((pb0, semp0), (pb1, semp1))
        for b, (pb, semp) in enumerate(rings):
            @pl.when(lo + b < hi)
            def _(b=b, pb=pb, semp=semp):
                fetch(lo + b, pb, semp)

        nk2 = (hi - lo + 1) // 2

        def panel_iter(k2, carry):
            for b, (pb, semp) in enumerate(rings):
                p = lo + 2 * k2 + b

                @pl.when(p < hi)
                def _(p=p, pb=pb, semp=semp):
                    drain_panel(pb, semp)
                    hits_for(p, pb, PANEL)

                    @pl.when(p + 2 < hi)
                    def _():
                        fetch(p + 2, pb, semp)
            return carry

        lax.fori_loop(0, nk2, panel_iter, 0)

        @pl.when(is_last)
        def _():
            hits_for(NFULL, tb, DIM)


def _dot_body(u_st, i_st, w64, b16, out,
              urows, irows, w_v, b_v, out_v, tr_v):
    wid = lax.axis_index("s") * _NC + lax.axis_index("c")
    base = wid * _BPW * DIM

    pltpu.sync_copy(u_st.at[pl.ds(base, _BPW * DIM)], urows)
    pltpu.sync_copy(i_st.at[pl.ds(base, _BPW * DIM)], irows)
    pltpu.sync_copy(w64, w_v)
    pltpu.sync_copy(b16, b_v)

    wvs = [w_v[pl.ds(j * LANES, LANES)] for j in range(DIM // LANES)]
    bv = b_v[...]
    scat_idx = lax.iota(jnp.int32, LANES) * LANES

    def group(g, carry):
        for l in range(LANES):
            b = g * LANES + l
            acc = jnp.zeros((LANES,), jnp.float32)
            for j in range(DIM // LANES):
                uv = urows[pl.ds(b * DIM + j * LANES, LANES)]
                iv = irows[pl.ds(b * DIM + j * LANES, LANES)]
                acc = acc + uv * iv * wvs[j]
            plsc.store_scatter(tr_v, [scat_idx + l], acc)
        tot = tr_v[pl.ds(0, LANES)]
        for l in range(1, LANES):
            tot = tot + tr_v[pl.ds(l * LANES, LANES)]
        x = tot + bv
        out_v[pl.ds(g * LANES, LANES)] = 1.0 / (1.0 + jnp.exp(-x))
        return carry

    lax.fori_loop(0, _NGROUP, group, 0)

    pltpu.sync_copy(out_v, out.at[pl.ds(wid * _BPW, _BPW)])


@jax.jit
def _gmf_sc(user, item, utT, itT, utail, itail, w64, b16):
    mesh = plsc.VectorSubcoreMesh(core_axis_name="c", subcore_axis_name="s")
    gather = functools.partial(
        pl.kernel,
        mesh=mesh,
        out_type=(jax.ShapeDtypeStruct((BATCH * DIM,), jnp.float32),
                  jax.ShapeDtypeStruct((BATCH * DIM,), jnp.float32)),
        scratch_types=[
            pltpu.VMEM((BATCH + LANES,), jnp.int32),
            pltpu.VMEM((_HCAP,), jnp.int32),
            pltpu.VMEM((_HCAP,), jnp.int32),
            pltpu.VMEM((DIM, PANEL), jnp.float32),
            pltpu.VMEM((DIM, PANEL), jnp.float32),
            pltpu.VMEM((DIM, DIM), jnp.float32),
            pltpu.VMEM((2 * LANES, DIM), jnp.float32),
            pltpu.SMEM((32,), jnp.int32),
            pltpu.SemaphoreType.DMA,
            pltpu.SemaphoreType.DMA,
            pltpu.SemaphoreType.DMA,
        ],
        compiler_params=pltpu.CompilerParams(needs_layout_passes=False),
    )(_gather_body)
    u_st, i_st = gather(user, item, utT, itT, utail, itail)

    dot = functools.partial(
        pl.kernel,
        mesh=mesh,
        out_type=jax.ShapeDtypeStruct((BATCH,), jnp.float32),
        scratch_types=[
            pltpu.VMEM((_BPW * DIM,), jnp.float32),
            pltpu.VMEM((_BPW * DIM,), jnp.float32),
            pltpu.VMEM((DIM,), jnp.float32),
            pltpu.VMEM((LANES,), jnp.float32),
            pltpu.VMEM((_BPW,), jnp.float32),
            pltpu.VMEM((LANES * LANES,), jnp.float32),
        ],
        compiler_params=pltpu.CompilerParams(needs_layout_passes=False),
    )(_dot_body)
    return dot(u_st, i_st, w64, b16)


def kernel(user, item, user_table, item_table, dense_w, dense_b):
    utT = user_table.T
    itT = item_table.T
    utail = user_table[TAILS:TAILS + DIM].T
    itail = item_table[TAILS:TAILS + DIM].T
    w64 = dense_w.reshape(DIM)
    b16 = jnp.broadcast_to(dense_b, (LANES,))
    return _gmf_sc(user.astype(jnp.int32), item.astype(jnp.int32),
                   utT, itT, utail, itail, w64, b16)
